# baseline (speedup 1.0000x reference)
import jax
import jax.numpy as jnp
from jax import lax
from jax.experimental import pallas as pl
from jax.experimental.pallas import tpu as pltpu

R, N = 64, 32768
LANES = 128
BNC = 4096
NC = N // BNC
CHUNKS = BNC // LANES
U = 2
CG = 16
TG = CHUNKS // CG

_INT_MAX = 2**31 - 1


def _tc_body(x_hbm, o_ref, buf0, buf1, sem0, sem1):
    bufs = (buf0, buf1)
    sems = (sem0, sem1)

    def copy(c):
        return pltpu.make_async_copy(
            x_hbm.at[:, pl.ds(c * BNC, BNC)], bufs[c % 2], sems[c % 2])

    copy(0).start()
    rvs = [jnp.full((R, LANES), -jnp.inf, jnp.float32) for _ in range(U)]
    ris = [jnp.zeros((R, LANES), jnp.int32) for _ in range(U)]

    for c in range(NC):
        if c + 1 < NC:
            copy(c + 1).start()
        copy(c).wait()
        buf = bufs[c % 2]

        def group(t, carry, c=c, buf=buf):
            rvs, ris = carry
            rvs, ris = list(rvs), list(ris)
            base = t * CG
            for jj in range(CG):
                k = jj % U
                chunk = buf[:, pl.ds((base + jj) * LANES, LANES)]
                m = chunk > rvs[k]
                rvs[k] = jnp.where(m, chunk, rvs[k])
                ris[k] = jnp.where(m, c * CHUNKS + base + jj, ris[k])
            return tuple(rvs), tuple(ris)

        rvs, ris = lax.fori_loop(0, TG, group, (tuple(rvs), tuple(ris)))
        rvs, ris = list(rvs), list(ris)

    lane = lax.broadcasted_iota(jnp.int32, (R, LANES), 1)
    pairs = [(rvs[k], ris[k] * LANES + lane) for k in range(U)]
    while len(pairs) > 1:
        nxt = []
        for a in range(0, len(pairs), 2):
            (va, ia), (vb, ib) = pairs[a], pairs[a + 1]
            take_b = (vb > va) | ((vb == va) & (ib < ia))
            nxt.append((jnp.where(take_b, vb, va),
                        jnp.where(take_b, ib, ia)))
        pairs = nxt
    rv, ri = pairs[0]
    mx = jnp.max(rv, axis=1, keepdims=True)
    cand = jnp.where(rv == mx, ri, _INT_MAX)
    o_ref[...] = jnp.min(cand, axis=1)[None, :]


@jax.jit
def _argmax_rows(x):
    out = pl.pallas_call(
        _tc_body,
        in_specs=[pl.BlockSpec(memory_space=pltpu.HBM)],
        out_specs=pl.BlockSpec((1, R), lambda: (0, 0)),
        out_shape=jax.ShapeDtypeStruct((1, R), jnp.int32),
        scratch_shapes=[
            pltpu.VMEM((R, BNC), jnp.float32),
            pltpu.VMEM((R, BNC), jnp.float32),
            pltpu.SemaphoreType.DMA,
            pltpu.SemaphoreType.DMA,
        ],
    )(x)
    return out.reshape(R)


def kernel(x):
    return _argmax_rows(x)
